# Initial kernel scaffold; baseline (speedup 1.0000x reference)
import jax
import jax.numpy as jnp
from jax.experimental import pallas as pl


def kernel(indices, table):
    raise NotImplementedError("write your pallas kernel here")


# speedup vs baseline: 106.4111x; 106.4111x over previous
import jax
import jax.numpy as jnp
from jax import lax
from jax.experimental import pallas as pl
from jax.experimental.pallas import tpu as pltpu
from jax.experimental.pallas import tpu_sc as plsc

VOCAB = 100000
D = 64
N = 4096 * 50
LANES = 128
ROWS = 784
VPAD = ROWS * LANES
NC = 2
NS = 16
NW = NC * NS
L = 16
CHUNK = N // NW
ZSL = VPAD // NS
GCH = 128
NGCH = CHUNK // GCH
PRE = 4

_mesh = plsc.VectorSubcoreMesh(core_axis_name="c", subcore_axis_name="s")


def _k1_body(idx_hbm, counts_hbm, shared, idx_v, ones_v, zeros_v):
    c = lax.axis_index("c")
    s = lax.axis_index("s")
    w = c * NS + s

    @pl.loop(0, ZSL // L)
    def _zero(i):
        zeros_v[pl.ds(i * L, L)] = jnp.zeros((L,), jnp.int32)

    @pl.loop(0, CHUNK // L)
    def _one(i):
        ones_v[pl.ds(i * L, L)] = jnp.ones((L,), jnp.int32)

    pltpu.sync_copy(zeros_v, shared.at[pl.ds(s * ZSL, ZSL)])
    plsc.subcore_barrier()
    pltpu.sync_copy(idx_hbm.at[pl.ds(w * CHUNK, CHUNK)], idx_v)
    pltpu.sync_copy(ones_v, shared.at[idx_v], add=True)
    plsc.subcore_barrier()
    pltpu.sync_copy(shared.at[pl.ds(s * ZSL, ZSL)],
                    counts_hbm.at[c, pl.ds(s * ZSL, ZSL)])


_k1 = pl.kernel(
    _k1_body,
    out_type=jax.ShapeDtypeStruct((NC, VPAD), jnp.int32),
    mesh=_mesh,
    compiler_params=pltpu.CompilerParams(use_tc_tiling_on_sc=False),
    scratch_types=[
        pltpu.VMEM_SHARED((VPAD,), jnp.int32),
        pltpu.VMEM((CHUNK,), jnp.int32),
        pltpu.VMEM((CHUNK,), jnp.int32),
        pltpu.VMEM((ZSL,), jnp.int32),
    ],
)


def _k2_body(counts_ref, rank_ref):
    cnt = counts_ref[0] + counts_ref[1]
    x = (cnt > 0).astype(jnp.float32)
    ci = lax.broadcasted_iota(jnp.int32, (LANES, LANES), 0)
    cj = lax.broadcasted_iota(jnp.int32, (LANES, LANES), 1)
    upper = (ci <= cj).astype(jnp.float32)
    prefix = jnp.dot(x, upper, preferred_element_type=jnp.float32)
    row_tot = prefix[:, LANES - 1:LANES]
    ri = lax.broadcasted_iota(jnp.int32, (ROWS, ROWS), 0)
    rj = lax.broadcasted_iota(jnp.int32, (ROWS, ROWS), 1)
    strict = (rj < ri).astype(jnp.float32)
    row_off = jnp.dot(strict, row_tot, preferred_element_type=jnp.float32)
    rank_ref[...] = (row_off + prefix - x).astype(jnp.int32)


_k2 = pl.pallas_call(
    _k2_body,
    out_shape=jax.ShapeDtypeStruct((ROWS, LANES), jnp.int32),
)


def _k3_body(rank_hbm, idx_hbm, table_hbm, out_hbm,
             idx_v, int_v, rows_v, sem_i, sem0, sem1):
    c = lax.axis_index("c")
    s = lax.axis_index("s")
    w = c * NS + s
    pltpu.sync_copy(idx_hbm.at[w], idx_v)

    tab_sems = (sem0, sem1)

    def fire_int(j):
        return pltpu.async_copy(rank_hbm.at[idx_v.at[j]], int_v.at[j], sem_i)

    def fire_tab(j):
        return pltpu.async_copy(table_hbm.at[int_v.at[j]],
                                rows_v.at[j % 2], tab_sems[j % 2])

    int_d = [None] * NGCH
    tab_d = [None] * NGCH
    for j in range(min(PRE, NGCH)):
        int_d[j] = fire_int(j)
    for j in range(NGCH):
        int_d[j].wait()
        tab_d[j] = fire_tab(j)
        if j + PRE < NGCH:
            int_d[j + PRE] = fire_int(j + PRE)
        if j >= 1:
            tab_d[j - 1].wait()
            pltpu.sync_copy(rows_v.at[(j - 1) % 2],
                            out_hbm.at[pl.ds(w * CHUNK + (j - 1) * GCH, GCH), :])
    tab_d[NGCH - 1].wait()
    pltpu.sync_copy(rows_v.at[(NGCH - 1) % 2],
                    out_hbm.at[pl.ds(w * CHUNK + (NGCH - 1) * GCH, GCH), :])


_k3 = pl.kernel(
    _k3_body,
    out_type=jax.ShapeDtypeStruct((N, D), jnp.float32),
    mesh=_mesh,
    compiler_params=pltpu.CompilerParams(use_tc_tiling_on_sc=False),
    scratch_types=[
        pltpu.VMEM((NGCH, GCH), jnp.int32),
        pltpu.VMEM((NGCH, GCH), jnp.int32),
        pltpu.VMEM((2, GCH, D), jnp.float32),
        pltpu.SemaphoreType.DMA,
        pltpu.SemaphoreType.DMA,
        pltpu.SemaphoreType.DMA,
    ],
)


def kernel(indices, table):
    idx_flat = indices.reshape(-1).astype(jnp.int32)
    counts = _k1(idx_flat)
    rank = _k2(counts.reshape(NC, ROWS, LANES))
    out = _k3(rank.reshape(-1), idx_flat.reshape(NW, NGCH, GCH), table)
    return out.reshape(indices.shape + (D,))


# speedup vs baseline: 107.9820x; 1.0148x over previous
import jax
import jax.numpy as jnp
from jax import lax
from jax.experimental import pallas as pl
from jax.experimental.pallas import tpu as pltpu
from jax.experimental.pallas import tpu_sc as plsc

VOCAB = 100000
D = 64
N = 4096 * 50
LANES = 128
ROWS = 784
VPAD = ROWS * LANES
NC = 2
NS = 16
NW = NC * NS
L = 16
CHUNK = N // NW
ZSL = VPAD // NS
B = 4096
H = 50
GCH = 800
NGCH = CHUNK // GCH
BROW = GCH // H
BROWS = CHUNK // H
PRE = 3

_mesh = plsc.VectorSubcoreMesh(core_axis_name="c", subcore_axis_name="s")


def _k1_body(idx_hbm, counts_hbm, shared, idx_v, ones_v, zeros_v):
    c = lax.axis_index("c")
    s = lax.axis_index("s")
    w = c * NS + s

    @pl.loop(0, ZSL // L)
    def _zero(i):
        zeros_v[pl.ds(i * L, L)] = jnp.zeros((L,), jnp.int32)

    @pl.loop(0, CHUNK // L)
    def _one(i):
        ones_v[pl.ds(i * L, L)] = jnp.ones((L,), jnp.int32)

    pltpu.sync_copy(zeros_v, shared.at[pl.ds(s * ZSL, ZSL)])
    plsc.subcore_barrier()
    pltpu.sync_copy(idx_hbm.at[pl.ds(w * CHUNK, CHUNK)], idx_v)
    pltpu.sync_copy(ones_v, shared.at[idx_v], add=True)
    plsc.subcore_barrier()
    pltpu.sync_copy(shared.at[pl.ds(s * ZSL, ZSL)],
                    counts_hbm.at[c, pl.ds(s * ZSL, ZSL)])


_k1 = pl.kernel(
    _k1_body,
    out_type=jax.ShapeDtypeStruct((NC, VPAD), jnp.int32),
    mesh=_mesh,
    compiler_params=pltpu.CompilerParams(use_tc_tiling_on_sc=False),
    scratch_types=[
        pltpu.VMEM_SHARED((VPAD,), jnp.int32),
        pltpu.VMEM((CHUNK,), jnp.int32),
        pltpu.VMEM((CHUNK,), jnp.int32),
        pltpu.VMEM((ZSL,), jnp.int32),
    ],
)


def _k2_body(counts_ref, rank_ref):
    cnt = counts_ref[0] + counts_ref[1]
    x = (cnt > 0).astype(jnp.float32)
    ci = lax.broadcasted_iota(jnp.int32, (LANES, LANES), 0)
    cj = lax.broadcasted_iota(jnp.int32, (LANES, LANES), 1)
    upper = (ci <= cj).astype(jnp.float32)
    prefix = jnp.dot(x, upper, preferred_element_type=jnp.float32)
    row_tot = prefix[:, LANES - 1:LANES]
    ri = lax.broadcasted_iota(jnp.int32, (ROWS, ROWS), 0)
    rj = lax.broadcasted_iota(jnp.int32, (ROWS, ROWS), 1)
    strict = (rj < ri).astype(jnp.float32)
    row_off = jnp.dot(strict, row_tot, preferred_element_type=jnp.float32)
    rank_ref[...] = (row_off + prefix - x).astype(jnp.int32)


_k2 = pl.pallas_call(
    _k2_body,
    out_shape=jax.ShapeDtypeStruct((ROWS, LANES), jnp.int32),
)


def _k3_body(rank_hbm, idx_hbm, table_hbm, out_hbm,
             idx_v, int_v, rows_v,
             sem_i0, sem_i1, sem_i2, sem_t0, sem_t1, sem_w0, sem_w1):
    c = lax.axis_index("c")
    s = lax.axis_index("s")
    w = c * NS + s
    pltpu.sync_copy(idx_hbm.at[w], idx_v)

    int_sems = (sem_i0, sem_i1, sem_i2)
    tab_sems = (sem_t0, sem_t1)
    wr_sems = (sem_w0, sem_w1)

    def fire_int(j):
        return pltpu.async_copy(rank_hbm.at[idx_v.at[j]], int_v.at[j],
                                int_sems[j % PRE])

    def fire_tab(j):
        return pltpu.async_copy(table_hbm.at[int_v.at[j]],
                                rows_v.at[j % 2], tab_sems[j % 2])

    def fire_writes(j):
        b = j % 2
        return [
            pltpu.async_copy(rows_v.at[b, pl.ds(i * H, H), :],
                             out_hbm.at[w * BROWS + j * BROW + i],
                             wr_sems[b])
            for i in range(BROW)
        ]

    int_d = [None] * NGCH
    tab_d = [None] * NGCH
    wr_d = [None] * NGCH
    for j in range(min(PRE, NGCH)):
        int_d[j] = fire_int(j)
    for j in range(NGCH):
        int_d[j].wait()
        if j >= 2:
            for d in wr_d[j - 2]:
                d.wait()
        tab_d[j] = fire_tab(j)
        if j + PRE < NGCH:
            int_d[j + PRE] = fire_int(j + PRE)
        if j >= 1:
            tab_d[j - 1].wait()
            wr_d[j - 1] = fire_writes(j - 1)
    tab_d[NGCH - 1].wait()
    wr_d[NGCH - 1] = fire_writes(NGCH - 1)
    for d in wr_d[NGCH - 2]:
        d.wait()
    for d in wr_d[NGCH - 1]:
        d.wait()


_k3 = pl.kernel(
    _k3_body,
    out_type=jax.ShapeDtypeStruct((B, H, D), jnp.float32),
    mesh=_mesh,
    compiler_params=pltpu.CompilerParams(use_tc_tiling_on_sc=False),
    scratch_types=[
        pltpu.VMEM((NGCH, GCH), jnp.int32),
        pltpu.VMEM((NGCH, GCH), jnp.int32),
        pltpu.VMEM((2, GCH, D), jnp.float32),
        pltpu.SemaphoreType.DMA,
        pltpu.SemaphoreType.DMA,
        pltpu.SemaphoreType.DMA,
        pltpu.SemaphoreType.DMA,
        pltpu.SemaphoreType.DMA,
        pltpu.SemaphoreType.DMA,
        pltpu.SemaphoreType.DMA,
    ],
)


def kernel(indices, table):
    idx_flat = indices.reshape(-1).astype(jnp.int32)
    counts = _k1(idx_flat)
    rank = _k2(counts.reshape(NC, ROWS, LANES))
    return _k3(rank.reshape(-1), idx_flat.reshape(NW, NGCH, GCH), table)


# speedup vs baseline: 127.3015x; 1.1789x over previous
import jax
import jax.numpy as jnp
from jax import lax
from jax.experimental import pallas as pl
from jax.experimental.pallas import tpu as pltpu
from jax.experimental.pallas import tpu_sc as plsc

VOCAB = 100000
D = 64
N = 4096 * 50
LANES = 128
ROWS = 784
VPAD = ROWS * LANES
NC = 2
NS = 16
NW = NC * NS
L = 16
CHUNK = N // NW
ZSL = VPAD // NS
B = 4096
H = 50
GCH = 800
NGCH = CHUNK // GCH
BROW = GCH // H
BROWS = CHUNK // H
PRE = 3

_mesh = plsc.VectorSubcoreMesh(core_axis_name="c", subcore_axis_name="s")


def _k1_body(idx_hbm, counts_hbm, shared, idx_v, ones_v, zeros_v):
    c = lax.axis_index("c")
    s = lax.axis_index("s")
    w = c * NS + s

    @pl.loop(0, ZSL // L)
    def _zero(i):
        zeros_v[pl.ds(i * L, L)] = jnp.zeros((L,), jnp.int32)

    @pl.loop(0, CHUNK // L)
    def _one(i):
        ones_v[pl.ds(i * L, L)] = jnp.ones((L,), jnp.int32)

    pltpu.sync_copy(zeros_v, shared.at[pl.ds(s * ZSL, ZSL)])
    plsc.subcore_barrier()
    pltpu.sync_copy(idx_hbm.at[pl.ds(w * CHUNK, CHUNK)], idx_v)
    pltpu.sync_copy(ones_v, shared.at[idx_v], add=True)
    plsc.subcore_barrier()
    pltpu.sync_copy(shared.at[pl.ds(s * ZSL, ZSL)],
                    counts_hbm.at[c, pl.ds(s * ZSL, ZSL)])


_k1 = pl.kernel(
    _k1_body,
    out_type=jax.ShapeDtypeStruct((NC, VPAD), jnp.int32),
    mesh=_mesh,
    compiler_params=pltpu.CompilerParams(use_tc_tiling_on_sc=False),
    scratch_types=[
        pltpu.VMEM_SHARED((VPAD,), jnp.int32),
        pltpu.VMEM((CHUNK,), jnp.int32),
        pltpu.VMEM((CHUNK,), jnp.int32),
        pltpu.VMEM((ZSL,), jnp.int32),
    ],
)


def _k2_body(counts_ref, rank_ref):
    cnt = counts_ref[0] + counts_ref[1]
    x = (cnt > 0).astype(jnp.float32)
    ci = lax.broadcasted_iota(jnp.int32, (LANES, LANES), 0)
    cj = lax.broadcasted_iota(jnp.int32, (LANES, LANES), 1)
    upper = (ci <= cj).astype(jnp.float32)
    prefix = jnp.dot(x, upper, preferred_element_type=jnp.float32)
    row_tot = prefix[:, LANES - 1:LANES]
    ri = lax.broadcasted_iota(jnp.int32, (ROWS, ROWS), 0)
    rj = lax.broadcasted_iota(jnp.int32, (ROWS, ROWS), 1)
    strict = (rj < ri).astype(jnp.float32)
    row_off = jnp.dot(strict, row_tot, preferred_element_type=jnp.float32)
    rank_ref[...] = (row_off + prefix - x).astype(jnp.int32)


_k2 = pl.pallas_call(
    _k2_body,
    out_shape=jax.ShapeDtypeStruct((ROWS, LANES), jnp.int32),
)


def _k3_body(rank_hbm, idx_hbm, table_hbm, out_hbm,
             idx_v, int_v, rows_v,
             sem_i0, sem_i1, sem_i2, sem_t0, sem_t1, sem_w0, sem_w1):
    c = lax.axis_index("c")
    s = lax.axis_index("s")
    w = c * NS + s
    pltpu.sync_copy(idx_hbm.at[w], idx_v)

    int_sems = (sem_i0, sem_i1, sem_i2)
    tab_sems = (sem_t0, sem_t1)
    wr_sems = (sem_w0, sem_w1)

    def fire_int(j):
        return pltpu.async_copy(rank_hbm.at[idx_v.at[j]], int_v.at[j],
                                int_sems[j % PRE])

    def fire_tab(j):
        return pltpu.async_copy(table_hbm.at[int_v.at[j]],
                                rows_v.at[j % 2], tab_sems[j % 2])

    def fire_writes(j):
        b = j % 2
        return [
            pltpu.async_copy(rows_v.at[b],
                             out_hbm.at[pl.ds(w * CHUNK + j * GCH, GCH), :],
                             wr_sems[b])
        ]

    int_d = [None] * NGCH
    tab_d = [None] * NGCH
    wr_d = [None] * NGCH
    for j in range(min(PRE, NGCH)):
        int_d[j] = fire_int(j)
    for j in range(NGCH):
        int_d[j].wait()
        if j >= 2:
            for d in wr_d[j - 2]:
                d.wait()
        tab_d[j] = fire_tab(j)
        if j + PRE < NGCH:
            int_d[j + PRE] = fire_int(j + PRE)
        if j >= 1:
            tab_d[j - 1].wait()
            wr_d[j - 1] = fire_writes(j - 1)
    tab_d[NGCH - 1].wait()
    wr_d[NGCH - 1] = fire_writes(NGCH - 1)
    for d in wr_d[NGCH - 2]:
        d.wait()
    for d in wr_d[NGCH - 1]:
        d.wait()


_k3 = pl.kernel(
    _k3_body,
    out_type=jax.ShapeDtypeStruct((N, D), jnp.float32),
    mesh=_mesh,
    compiler_params=pltpu.CompilerParams(use_tc_tiling_on_sc=False),
    scratch_types=[
        pltpu.VMEM((NGCH, GCH), jnp.int32),
        pltpu.VMEM((NGCH, GCH), jnp.int32),
        pltpu.VMEM((2, GCH, D), jnp.float32),
        pltpu.SemaphoreType.DMA,
        pltpu.SemaphoreType.DMA,
        pltpu.SemaphoreType.DMA,
        pltpu.SemaphoreType.DMA,
        pltpu.SemaphoreType.DMA,
        pltpu.SemaphoreType.DMA,
        pltpu.SemaphoreType.DMA,
    ],
)


BB = 128
NB = B // BB
INROWS = BB * H * D // LANES


def _k4_body(in_ref, out_ref):
    x = in_ref[...]
    y = x.reshape(BB, H // 2, LANES)
    z0 = y[:, :, :D]
    z1 = y[:, :, D:]
    t0 = jnp.transpose(z0, (1, 2, 0))
    t1 = jnp.transpose(z1, (1, 2, 0))
    st = jnp.stack([t0, t1], axis=1)
    out_ref[...] = st.reshape(H, D, BB)


_k4 = pl.pallas_call(
    _k4_body,
    grid=(NB,),
    in_specs=[pl.BlockSpec((INROWS, LANES), lambda i: (i, 0))],
    out_specs=pl.BlockSpec((H, D, BB), lambda i: (0, 0, i)),
    out_shape=jax.ShapeDtypeStruct((H, D, B), jnp.float32),
)


def kernel(indices, table):
    idx_flat = indices.reshape(-1).astype(jnp.int32)
    counts = _k1(idx_flat)
    rank = _k2(counts.reshape(NC, ROWS, LANES))
    gathered = _k3(rank.reshape(-1), idx_flat.reshape(NW, NGCH, GCH), table)
    out_t = _k4(gathered.reshape(N * D // LANES, LANES))
    return jnp.transpose(out_t, (2, 0, 1))
